# initial kernel scaffold (unmeasured)
import jax
import jax.numpy as jnp
from jax import lax
from jax.experimental import pallas as pl
from jax.experimental.pallas import tpu as pltpu

N_DEV = 4


def kernel(x, w_mat):
    m, k_shard = x.shape
    _, n = w_mat.shape
    m_out = m // N_DEV

    def body(x_ref, w_ref, out_ref,
             send_buf, recv_bufs, send_sems, recv_sems,
             my_amax, peer_amax, amax_send_sems, amax_recv_sems):
        d = lax.axis_index("i")
        left = (d - 1) % N_DEV
        right = (d + 1) % N_DEV

        peer_amax[...] = jnp.zeros((N_DEV, 8, 128), jnp.float32)

        barrier_sem = pltpu.get_barrier_semaphore()
        for nbr in [left, right]:
            pl.semaphore_signal(
                barrier_sem, inc=1,
                device_id=(nbr,), device_id_type=pl.DeviceIdType.MESH,
            )
        pl.semaphore_wait(barrier_sem, 2)

        def chunk_gemm(idx):
            xs = x_ref[pl.ds(idx * m_out, m_out), :].astype(jnp.bfloat16)
            ws = w_ref[...].astype(jnp.bfloat16)
            return jnp.dot(xs, ws, preferred_element_type=jnp.float32)

        for h in range(N_DEV - 1):
            idx = (d + (N_DEV - 1) - h) % N_DEV
            val = chunk_gemm(idx)
            if h > 0:
                val = val + recv_bufs[h - 1].astype(jnp.float32)
            send_buf[...] = val.astype(jnp.bfloat16)
            rdma = pltpu.make_async_remote_copy(
                src_ref=send_buf,
                dst_ref=recv_bufs.at[h],
                send_sem=send_sems.at[h],
                recv_sem=recv_sems.at[h],
                device_id=(right,),
                device_id_type=pl.DeviceIdType.MESH,
            )
            rdma.start()
            rdma.wait()

        y = chunk_gemm(d) + recv_bufs[N_DEV - 2].astype(jnp.float32)
        out_ref[...] = y

        local_amax = jnp.max(jnp.abs(y))
        my_amax[...] = jnp.full((8, 128), local_amax, jnp.float32)
        peers = [right, (d + 2) % N_DEV, left]
        rdmas = []
        for p in peers:
            r = pltpu.make_async_remote_copy(
                src_ref=my_amax,
                dst_ref=peer_amax.at[d],
                send_sem=amax_send_sems.at[p],
                recv_sem=amax_recv_sems.at[d],
                device_id=(p,),
                device_id_type=pl.DeviceIdType.MESH,
            )
            r.start()
            rdmas.append(r)
        for r in rdmas:
            r.wait_send()
        for p in peers:
            recv = pltpu.make_async_remote_copy(
                src_ref=my_amax,
                dst_ref=peer_amax.at[p],
                send_sem=amax_send_sems.at[p],
                recv_sem=amax_recv_sems.at[p],
                device_id=(p,),
                device_id_type=pl.DeviceIdType.MESH,
            )
            recv.wait_recv()
        global_amax = jnp.maximum(local_amax, jnp.max(peer_amax[...]))

        scale = global_amax / 127.0
        yq = out_ref[...]
        q = jnp.clip(jnp.round(yq / scale), -127.0, 127.0)
        out_ref[...] = q * scale

    return pl.pallas_call(
        body,
        out_shape=jax.ShapeDtypeStruct((m_out, n), jnp.float32),
        in_specs=[
            pl.BlockSpec(memory_space=pltpu.VMEM),
            pl.BlockSpec(memory_space=pltpu.VMEM),
        ],
        out_specs=pl.BlockSpec(memory_space=pltpu.VMEM),
        scratch_shapes=[
            pltpu.VMEM((m_out, n), jnp.bfloat16),
            pltpu.VMEM((N_DEV - 1, m_out, n), jnp.bfloat16),
            pltpu.SemaphoreType.DMA((N_DEV - 1,)),
            pltpu.SemaphoreType.DMA((N_DEV - 1,)),
            pltpu.VMEM((8, 128), jnp.float32),
            pltpu.VMEM((N_DEV, 8, 128), jnp.float32),
            pltpu.SemaphoreType.DMA((N_DEV,)),
            pltpu.SemaphoreType.DMA((N_DEV,)),
        ],
        compiler_params=pltpu.CompilerParams(collective_id=0),
    )(x, w_mat)


# baseline (device time: 185794 ns/iter reference)
import jax
import jax.numpy as jnp
from jax import lax
from jax.experimental import pallas as pl
from jax.experimental.pallas import tpu as pltpu

N_DEV = 4


def kernel(x, w_mat):
    m, k_shard = x.shape
    _, n = w_mat.shape
    m_out = m // N_DEV

    def body(x_ref, w_ref, out_ref,
             send_buf, recv_bufs, send_sems, recv_sems,
             my_amax, peer_amax, amax_send_sems, amax_recv_sems):
        d = lax.axis_index("i")
        left = (d - 1) % N_DEV
        right = (d + 1) % N_DEV

        peer_amax[...] = jnp.zeros((N_DEV, 8, 128), jnp.float32)

        barrier_sem = pltpu.get_barrier_semaphore()
        for nbr in [left, right]:
            pl.semaphore_signal(
                barrier_sem, inc=1,
                device_id=(nbr,), device_id_type=pl.DeviceIdType.MESH,
            )
        pl.semaphore_wait(barrier_sem, 2)

        def chunk_gemm(idx):
            xs = x_ref[pl.ds(idx * m_out, m_out), :].astype(jnp.bfloat16)
            ws = w_ref[...].astype(jnp.bfloat16)
            return jnp.dot(xs, ws, preferred_element_type=jnp.float32)

        for h in range(N_DEV - 1):
            idx = (d + (N_DEV - 1) - h) % N_DEV
            val = chunk_gemm(idx)
            if h > 0:
                val = val + recv_bufs[h - 1].astype(jnp.float32)
            send_buf[...] = val.astype(jnp.bfloat16)
            rdma = pltpu.make_async_remote_copy(
                src_ref=send_buf,
                dst_ref=recv_bufs.at[h],
                send_sem=send_sems.at[h],
                recv_sem=recv_sems.at[h],
                device_id=(right,),
                device_id_type=pl.DeviceIdType.MESH,
            )
            rdma.start()
            rdma.wait()

        y = chunk_gemm(d) + recv_bufs[N_DEV - 2].astype(jnp.float32)
        out_ref[...] = y

        local_amax = jnp.max(jnp.abs(y))
        my_amax[...] = jnp.full((8, 128), local_amax, jnp.float32)
        peers = [right, (d + 2) % N_DEV, left]
        rdmas = []
        for p in peers:
            r = pltpu.make_async_remote_copy(
                src_ref=my_amax,
                dst_ref=peer_amax.at[d],
                send_sem=amax_send_sems.at[p],
                recv_sem=amax_recv_sems.at[d],
                device_id=(p,),
                device_id_type=pl.DeviceIdType.MESH,
            )
            r.start()
            rdmas.append(r)
        for r in rdmas:
            r.wait_send()
        for p in peers:
            recv = pltpu.make_async_remote_copy(
                src_ref=my_amax,
                dst_ref=peer_amax.at[p],
                send_sem=amax_send_sems.at[p],
                recv_sem=amax_recv_sems.at[p],
                device_id=(p,),
                device_id_type=pl.DeviceIdType.MESH,
            )
            recv.wait_recv()
        global_amax = jnp.maximum(local_amax, jnp.max(peer_amax[...]))

        scale = global_amax / 127.0
        yq = out_ref[...]
        q = jnp.clip(jnp.round(yq / scale), -127.0, 127.0)
        out_ref[...] = q * scale

    return pl.pallas_call(
        body,
        out_shape=jax.ShapeDtypeStruct((m_out, n), jnp.float32),
        in_specs=[
            pl.BlockSpec(memory_space=pltpu.VMEM),
            pl.BlockSpec(memory_space=pltpu.VMEM),
        ],
        out_specs=pl.BlockSpec(memory_space=pltpu.VMEM),
        scratch_shapes=[
            pltpu.VMEM((m_out, n), jnp.bfloat16),
            pltpu.VMEM((N_DEV - 1, m_out, n), jnp.bfloat16),
            pltpu.SemaphoreType.DMA((N_DEV - 1,)),
            pltpu.SemaphoreType.DMA((N_DEV - 1,)),
            pltpu.VMEM((8, 128), jnp.float32),
            pltpu.VMEM((N_DEV, 8, 128), jnp.float32),
            pltpu.SemaphoreType.DMA((N_DEV,)),
            pltpu.SemaphoreType.DMA((N_DEV,)),
        ],
        compiler_params=pltpu.CompilerParams(
            collective_id=0, vmem_limit_bytes=100 * 1024 * 1024
        ),
    )(x, w_mat)


# device time: 129776 ns/iter; 1.4317x vs baseline; 1.4317x over previous
import jax
import jax.numpy as jnp
from jax import lax
from jax.experimental import pallas as pl
from jax.experimental.pallas import tpu as pltpu

N_DEV = 4


def kernel(x, w_mat):
    m, k_shard = x.shape
    _, n = w_mat.shape
    m_out = m // N_DEV

    n_half = n // 2

    def body(x_ref, w_ref, out_ref,
             send_bufs, recv_bufs, send_sems, recv_sems,
             my_amax, peer_amax, amax_send_sems, amax_recv_sems):
        d = lax.axis_index("i")
        left = (d - 1) % N_DEV
        right = (d + 1) % N_DEV

        peer_amax[...] = jnp.zeros((N_DEV, 8, 128), jnp.float32)

        barrier_sem = pltpu.get_barrier_semaphore()
        for nbr in [left, right]:
            pl.semaphore_signal(
                barrier_sem, inc=1,
                device_id=(nbr,), device_id_type=pl.DeviceIdType.MESH,
            )
        pl.semaphore_wait(barrier_sem, 2)

        def hgemm(idx, half):
            xs = x_ref[pl.ds(idx * m_out, m_out), :]
            ws = w_ref[:, pl.ds(half * n_half, n_half)]
            return jnp.dot(xs, ws, preferred_element_type=jnp.float32)

        vr = hgemm((d + 3) % N_DEV, 0)
        vl = hgemm((d + 1) % N_DEV, 1)
        for h in range(N_DEV - 1):
            send_bufs[0, :, :] = vr.astype(jnp.bfloat16)
            send_bufs[1, :, :] = vl.astype(jnp.bfloat16)
            rdma_r = pltpu.make_async_remote_copy(
                src_ref=send_bufs.at[0],
                dst_ref=recv_bufs.at[h, 0],
                send_sem=send_sems.at[h, 0],
                recv_sem=recv_sems.at[h, 0],
                device_id=(right,),
                device_id_type=pl.DeviceIdType.MESH,
            )
            rdma_l = pltpu.make_async_remote_copy(
                src_ref=send_bufs.at[1],
                dst_ref=recv_bufs.at[h, 1],
                send_sem=send_sems.at[h, 1],
                recv_sem=recv_sems.at[h, 1],
                device_id=(left,),
                device_id_type=pl.DeviceIdType.MESH,
            )
            rdma_r.start()
            rdma_l.start()
            if h < N_DEV - 2:
                nr = hgemm((d + 2 - h) % N_DEV, 0)
                nl = hgemm((d + 2 + h) % N_DEV, 1)
            else:
                nr = hgemm(d, 0)
                nl = hgemm(d, 1)
            rdma_r.wait()
            rdma_l.wait()
            vr = nr + recv_bufs[h, 0].astype(jnp.float32)
            vl = nl + recv_bufs[h, 1].astype(jnp.float32)

        out_ref[:, pl.ds(0, n_half)] = vr
        out_ref[:, pl.ds(n_half, n_half)] = vl
        y = jnp.concatenate([vr, vl], axis=1)

        local_amax = jnp.max(jnp.abs(y))
        my_amax[...] = jnp.full((8, 128), local_amax, jnp.float32)
        peers = [right, (d + 2) % N_DEV, left]
        rdmas = []
        for p in peers:
            r = pltpu.make_async_remote_copy(
                src_ref=my_amax,
                dst_ref=peer_amax.at[d],
                send_sem=amax_send_sems.at[p],
                recv_sem=amax_recv_sems.at[d],
                device_id=(p,),
                device_id_type=pl.DeviceIdType.MESH,
            )
            r.start()
            rdmas.append(r)
        for r in rdmas:
            r.wait_send()
        for p in peers:
            recv = pltpu.make_async_remote_copy(
                src_ref=my_amax,
                dst_ref=peer_amax.at[p],
                send_sem=amax_send_sems.at[p],
                recv_sem=amax_recv_sems.at[p],
                device_id=(p,),
                device_id_type=pl.DeviceIdType.MESH,
            )
            recv.wait_recv()
        global_amax = jnp.maximum(local_amax, jnp.max(peer_amax[...]))

        scale = global_amax / 127.0
        yq = out_ref[...]
        q = jnp.clip(jnp.round(yq / scale), -127.0, 127.0)
        out_ref[...] = q * scale

    return pl.pallas_call(
        body,
        out_shape=jax.ShapeDtypeStruct((m_out, n), jnp.float32),
        in_specs=[
            pl.BlockSpec(memory_space=pltpu.VMEM),
            pl.BlockSpec(memory_space=pltpu.VMEM),
        ],
        out_specs=pl.BlockSpec(memory_space=pltpu.VMEM),
        scratch_shapes=[
            pltpu.VMEM((2, m_out, n // 2), jnp.bfloat16),
            pltpu.VMEM((N_DEV - 1, 2, m_out, n // 2), jnp.bfloat16),
            pltpu.SemaphoreType.DMA((N_DEV - 1, 2)),
            pltpu.SemaphoreType.DMA((N_DEV - 1, 2)),
            pltpu.VMEM((8, 128), jnp.float32),
            pltpu.VMEM((N_DEV, 8, 128), jnp.float32),
            pltpu.SemaphoreType.DMA((N_DEV,)),
            pltpu.SemaphoreType.DMA((N_DEV,)),
        ],
        compiler_params=pltpu.CompilerParams(
            collective_id=0, vmem_limit_bytes=100 * 1024 * 1024
        ),
    )(x.astype(jnp.bfloat16), w_mat.astype(jnp.bfloat16))


# device time: 122398 ns/iter; 1.5179x vs baseline; 1.0603x over previous
import jax
import jax.numpy as jnp
from jax import lax
from jax.experimental import pallas as pl
from jax.experimental.pallas import tpu as pltpu

N_DEV = 4


def kernel(x, w_mat):
    m, k_shard = x.shape
    _, n = w_mat.shape
    m_out = m // N_DEV

    n_half = n // 2

    def body(x_ref, w_ref, out_ref,
             send_bufs, recv_bufs, send_sems, recv_sems,
             my_amax, peer_amax, amax_send_sems, amax_recv_sems):
        d = lax.axis_index("i")
        left = (d - 1) % N_DEV
        right = (d + 1) % N_DEV

        peer_amax[...] = jnp.zeros((N_DEV, 8, 128), jnp.float32)

        barrier_sem = pltpu.get_barrier_semaphore()
        for nbr in [left, right]:
            pl.semaphore_signal(
                barrier_sem, inc=1,
                device_id=(nbr,), device_id_type=pl.DeviceIdType.MESH,
            )
        pl.semaphore_wait(barrier_sem, 2)

        def hgemm(idx, half):
            xs = x_ref[pl.ds(idx * m_out, m_out), :].astype(jnp.bfloat16)
            ws = w_ref[:, pl.ds(half * n_half, n_half)]
            return jnp.dot(xs, ws, preferred_element_type=jnp.float32)

        vr = hgemm((d + 3) % N_DEV, 0)
        vl = hgemm((d + 1) % N_DEV, 1)
        for h in range(N_DEV - 1):
            send_bufs[0, :, :] = vr.astype(jnp.bfloat16)
            send_bufs[1, :, :] = vl.astype(jnp.bfloat16)
            rdma_r = pltpu.make_async_remote_copy(
                src_ref=send_bufs.at[0],
                dst_ref=recv_bufs.at[h, 0],
                send_sem=send_sems.at[h, 0],
                recv_sem=recv_sems.at[h, 0],
                device_id=(right,),
                device_id_type=pl.DeviceIdType.MESH,
            )
            rdma_l = pltpu.make_async_remote_copy(
                src_ref=send_bufs.at[1],
                dst_ref=recv_bufs.at[h, 1],
                send_sem=send_sems.at[h, 1],
                recv_sem=recv_sems.at[h, 1],
                device_id=(left,),
                device_id_type=pl.DeviceIdType.MESH,
            )
            rdma_r.start()
            rdma_l.start()
            if h < N_DEV - 2:
                nr = hgemm((d + 2 - h) % N_DEV, 0)
                nl = hgemm((d + 2 + h) % N_DEV, 1)
            else:
                nr = hgemm(d, 0)
                nl = hgemm(d, 1)
            rdma_r.wait()
            rdma_l.wait()
            vr = nr + recv_bufs[h, 0].astype(jnp.float32)
            vl = nl + recv_bufs[h, 1].astype(jnp.float32)

        out_ref[:, pl.ds(0, n_half)] = vr
        out_ref[:, pl.ds(n_half, n_half)] = vl
        y = jnp.concatenate([vr, vl], axis=1)

        local_amax = jnp.max(jnp.abs(y))
        my_amax[...] = jnp.full((8, 128), local_amax, jnp.float32)
        peers = [right, (d + 2) % N_DEV, left]
        rdmas = []
        for p in peers:
            r = pltpu.make_async_remote_copy(
                src_ref=my_amax,
                dst_ref=peer_amax.at[d],
                send_sem=amax_send_sems.at[p],
                recv_sem=amax_recv_sems.at[d],
                device_id=(p,),
                device_id_type=pl.DeviceIdType.MESH,
            )
            r.start()
            rdmas.append(r)
        for r in rdmas:
            r.wait_send()
        for p in peers:
            recv = pltpu.make_async_remote_copy(
                src_ref=my_amax,
                dst_ref=peer_amax.at[p],
                send_sem=amax_send_sems.at[p],
                recv_sem=amax_recv_sems.at[p],
                device_id=(p,),
                device_id_type=pl.DeviceIdType.MESH,
            )
            recv.wait_recv()
        global_amax = jnp.maximum(local_amax, jnp.max(peer_amax[...]))

        scale = global_amax / 127.0
        yq = out_ref[...]
        q = jnp.clip(jnp.round(yq / scale), -127.0, 127.0)
        out_ref[...] = q * scale

    return pl.pallas_call(
        body,
        out_shape=jax.ShapeDtypeStruct((m_out, n), jnp.float32),
        in_specs=[
            pl.BlockSpec(memory_space=pltpu.VMEM),
            pl.BlockSpec(memory_space=pltpu.VMEM),
        ],
        out_specs=pl.BlockSpec(memory_space=pltpu.VMEM),
        scratch_shapes=[
            pltpu.VMEM((2, m_out, n // 2), jnp.bfloat16),
            pltpu.VMEM((N_DEV - 1, 2, m_out, n // 2), jnp.bfloat16),
            pltpu.SemaphoreType.DMA((N_DEV - 1, 2)),
            pltpu.SemaphoreType.DMA((N_DEV - 1, 2)),
            pltpu.VMEM((8, 128), jnp.float32),
            pltpu.VMEM((N_DEV, 8, 128), jnp.float32),
            pltpu.SemaphoreType.DMA((N_DEV,)),
            pltpu.SemaphoreType.DMA((N_DEV,)),
        ],
        compiler_params=pltpu.CompilerParams(
            collective_id=0, vmem_limit_bytes=100 * 1024 * 1024
        ),
    )(x, w_mat.astype(jnp.bfloat16))


# device time: 103222 ns/iter; 1.7999x vs baseline; 1.1858x over previous
import jax
import jax.numpy as jnp
from jax import lax
from jax.experimental import pallas as pl
from jax.experimental.pallas import tpu as pltpu

N_DEV = 4


def kernel(x, w_mat):
    m, k_shard = x.shape
    _, n = w_mat.shape
    m_out = m // N_DEV

    n_half = n // 2

    def body(x_ref, w_ref, out_ref,
             send_bufs, recv_bufs, send_sems, recv_sems,
             my_amax, peer_amax, amax_send_sems, amax_recv_sems):
        d = lax.axis_index("i")
        left = (d - 1) % N_DEV
        right = (d + 1) % N_DEV

        peer_amax[...] = jnp.zeros((N_DEV, 8, 128), jnp.float32)

        barrier_sem = pltpu.get_barrier_semaphore()
        for nbr in [left, right]:
            pl.semaphore_signal(
                barrier_sem, inc=1,
                device_id=(nbr,), device_id_type=pl.DeviceIdType.MESH,
            )
        pl.semaphore_wait(barrier_sem, 2)

        m_q = m_out // 2

        def qgemm(idx, dirn, s):
            xs = x_ref[pl.ds(idx * m_out + s * m_q, m_q), :].astype(jnp.bfloat16)
            ws = w_ref[:, pl.ds(dirn * n_half, n_half)]
            return jnp.dot(xs, ws, preferred_element_type=jnp.float32)

        SUBS = [(0, 0), (1, 0), (0, 1), (1, 1)]

        def mk(h, dirn, s):
            tgt = right if dirn == 0 else left
            return pltpu.make_async_remote_copy(
                src_ref=send_bufs.at[dirn, s],
                dst_ref=recv_bufs.at[h, dirn, s],
                send_sem=send_sems.at[h, dirn, s],
                recv_sem=recv_sems.at[h, dirn, s],
                device_id=(tgt,),
                device_id_type=pl.DeviceIdType.MESH,
            )

        start_idx = {0: (d + 3) % N_DEV, 1: (d + 1) % N_DEV}
        inflight = {}
        for dirn, s in SUBS:
            g0 = qgemm(start_idx[dirn], dirn, s)
            send_bufs[dirn, s] = g0.astype(jnp.bfloat16)
            r = mk(0, dirn, s)
            r.start()
            inflight[(dirn, s)] = r

        local_amax = jnp.float32(0.0)
        for h in range(N_DEV - 1):
            nidx = {0: (d + 2 - h) % N_DEV, 1: (d + 2 + h) % N_DEV}
            g = {}
            for dirn, s in SUBS:
                g[(dirn, s)] = qgemm(nidx[dirn], dirn, s)
            for dirn, s in SUBS:
                rd = inflight[(dirn, s)]
                rd.wait_recv()
                rd.wait_send()
                val = g[(dirn, s)] + recv_bufs[h, dirn, s].astype(jnp.float32)
                if h < N_DEV - 2:
                    send_bufs[dirn, s] = val.astype(jnp.bfloat16)
                    nxt = mk(h + 1, dirn, s)
                    nxt.start()
                    inflight[(dirn, s)] = nxt
                else:
                    out_ref[pl.ds(s * m_q, m_q), pl.ds(dirn * n_half, n_half)] = val
                    local_amax = jnp.maximum(local_amax, jnp.max(jnp.abs(val)))

        my_amax[...] = jnp.full((8, 128), local_amax, jnp.float32)
        peers = [right, (d + 2) % N_DEV, left]
        rdmas = []
        for p in peers:
            r = pltpu.make_async_remote_copy(
                src_ref=my_amax,
                dst_ref=peer_amax.at[d],
                send_sem=amax_send_sems.at[p],
                recv_sem=amax_recv_sems.at[d],
                device_id=(p,),
                device_id_type=pl.DeviceIdType.MESH,
            )
            r.start()
            rdmas.append(r)
        for r in rdmas:
            r.wait_send()
        for p in peers:
            recv = pltpu.make_async_remote_copy(
                src_ref=my_amax,
                dst_ref=peer_amax.at[p],
                send_sem=amax_send_sems.at[p],
                recv_sem=amax_recv_sems.at[p],
                device_id=(p,),
                device_id_type=pl.DeviceIdType.MESH,
            )
            recv.wait_recv()
        global_amax = jnp.maximum(local_amax, jnp.max(peer_amax[...]))

        scale = global_amax / 127.0
        yq = out_ref[...]
        q = jnp.clip(jnp.round(yq / scale), -127.0, 127.0)
        out_ref[...] = q * scale

    return pl.pallas_call(
        body,
        out_shape=jax.ShapeDtypeStruct((m_out, n), jnp.float32),
        in_specs=[
            pl.BlockSpec(memory_space=pltpu.VMEM),
            pl.BlockSpec(memory_space=pltpu.VMEM),
        ],
        out_specs=pl.BlockSpec(memory_space=pltpu.VMEM),
        scratch_shapes=[
            pltpu.VMEM((2, 2, m_out // 2, n // 2), jnp.bfloat16),
            pltpu.VMEM((N_DEV - 1, 2, 2, m_out // 2, n // 2), jnp.bfloat16),
            pltpu.SemaphoreType.DMA((N_DEV - 1, 2, 2)),
            pltpu.SemaphoreType.DMA((N_DEV - 1, 2, 2)),
            pltpu.VMEM((8, 128), jnp.float32),
            pltpu.VMEM((N_DEV, 8, 128), jnp.float32),
            pltpu.SemaphoreType.DMA((N_DEV,)),
            pltpu.SemaphoreType.DMA((N_DEV,)),
        ],
        compiler_params=pltpu.CompilerParams(
            collective_id=0, vmem_limit_bytes=100 * 1024 * 1024
        ),
    )(x, w_mat.astype(jnp.bfloat16))


# device time: 99566 ns/iter; 1.8660x vs baseline; 1.0367x over previous
import os

import jax
import jax.numpy as jnp
from jax import lax
from jax.experimental import pallas as pl
from jax.experimental.pallas import tpu as pltpu

N_DEV = 4
try:
    with open(os.path.join(os.path.dirname(__file__), "kdiag.txt")) as _f:
        _DIAG = _f.read().strip()
except OSError:
    _DIAG = ""


def kernel(x, w_mat):
    m, k_shard = x.shape
    _, n = w_mat.shape
    m_out = m // N_DEV

    n_half = n // 2

    def body(x_ref, w_ref, out_ref,
             send_bufs, recv_bufs, send_sems, recv_sems,
             my_amax, peer_amax, amax_send_sems, amax_recv_sems):
        d = lax.axis_index("i")
        left = (d - 1) % N_DEV
        right = (d + 1) % N_DEV

        peer_amax[...] = jnp.zeros((N_DEV, 8, 128), jnp.float32)

        barrier_sem = pltpu.get_barrier_semaphore()
        for nbr in [left, right]:
            pl.semaphore_signal(
                barrier_sem, inc=1,
                device_id=(nbr,), device_id_type=pl.DeviceIdType.MESH,
            )
        pl.semaphore_wait(barrier_sem, 2)

        m_q = m_out // 2

        def qgemm(idx, dirn, s):
            if _DIAG == "comm":
                return jnp.zeros((m_q, n_half), jnp.float32)
            xs = x_ref[pl.ds(idx * m_out + s * m_q, m_q), :].astype(jnp.bfloat16)
            ws = w_ref[:, pl.ds(dirn * n_half, n_half)]
            return jnp.dot(xs, ws, preferred_element_type=jnp.float32)

        SUBS = [(0, 0), (1, 0), (0, 1), (1, 1)]

        def mk(h, dirn, s):
            tgt = right if dirn == 0 else left
            return pltpu.make_async_remote_copy(
                src_ref=send_bufs.at[dirn, s],
                dst_ref=recv_bufs.at[h, dirn, s],
                send_sem=send_sems.at[h, dirn, s],
                recv_sem=recv_sems.at[h, dirn, s],
                device_id=(tgt,),
                device_id_type=pl.DeviceIdType.MESH,
            )

        start_idx = {0: (d + 3) % N_DEV, 1: (d + 1) % N_DEV}
        inflight = {}
        for dirn, s in SUBS:
            g0 = qgemm(start_idx[dirn], dirn, s)
            send_bufs[dirn, s] = g0.astype(jnp.bfloat16)
            if _DIAG != "compute":
                r = mk(0, dirn, s)
                r.start()
                inflight[(dirn, s)] = r

        local_amax = jnp.float32(0.0)
        for h in range(N_DEV - 1):
            nidx = {0: (d + 2 - h) % N_DEV, 1: (d + 2 + h) % N_DEV}
            g = {}
            for dirn, s in SUBS:
                g[(dirn, s)] = qgemm(nidx[dirn], dirn, s)
            for dirn, s in SUBS:
                if _DIAG != "compute":
                    rd = inflight[(dirn, s)]
                    rd.wait_recv()
                    rd.wait_send()
                val = g[(dirn, s)] + recv_bufs[h, dirn, s].astype(jnp.float32)
                if h < N_DEV - 2:
                    send_bufs[dirn, s] = val.astype(jnp.bfloat16)
                    if _DIAG != "compute":
                        nxt = mk(h + 1, dirn, s)
                        nxt.start()
                        inflight[(dirn, s)] = nxt
                else:
                    out_ref[pl.ds(s * m_q, m_q), pl.ds(dirn * n_half, n_half)] = val
                    local_amax = jnp.maximum(local_amax, jnp.max(jnp.abs(val)))

        my_amax[...] = jnp.full((8, 128), local_amax, jnp.float32)
        peers = [right, (d + 2) % N_DEV, left] if _DIAG != "compute" else []
        rdmas = []
        for p in peers:
            r = pltpu.make_async_remote_copy(
                src_ref=my_amax,
                dst_ref=peer_amax.at[d],
                send_sem=amax_send_sems.at[p],
                recv_sem=amax_recv_sems.at[d],
                device_id=(p,),
                device_id_type=pl.DeviceIdType.MESH,
            )
            r.start()
            rdmas.append(r)
        for r in rdmas:
            r.wait_send()
        for p in peers:
            recv = pltpu.make_async_remote_copy(
                src_ref=my_amax,
                dst_ref=peer_amax.at[p],
                send_sem=amax_send_sems.at[p],
                recv_sem=amax_recv_sems.at[p],
                device_id=(p,),
                device_id_type=pl.DeviceIdType.MESH,
            )
            recv.wait_recv()
        global_amax = jnp.maximum(local_amax, jnp.max(peer_amax[...]))

        scale = global_amax / 127.0
        yq = out_ref[...]
        q = jnp.clip(jnp.round(yq / scale), -127.0, 127.0)
        out_ref[...] = q * scale

    return pl.pallas_call(
        body,
        out_shape=jax.ShapeDtypeStruct((m_out, n), jnp.float32),
        in_specs=[
            pl.BlockSpec(memory_space=pltpu.VMEM),
            pl.BlockSpec(memory_space=pltpu.VMEM),
        ],
        out_specs=pl.BlockSpec(memory_space=pltpu.VMEM),
        scratch_shapes=[
            pltpu.VMEM((2, 2, m_out // 2, n // 2), jnp.bfloat16),
            pltpu.VMEM((N_DEV - 1, 2, 2, m_out // 2, n // 2), jnp.bfloat16),
            pltpu.SemaphoreType.DMA((N_DEV - 1, 2, 2)),
            pltpu.SemaphoreType.DMA((N_DEV - 1, 2, 2)),
            pltpu.VMEM((8, 128), jnp.float32),
            pltpu.VMEM((N_DEV, 8, 128), jnp.float32),
            pltpu.SemaphoreType.DMA((N_DEV,)),
            pltpu.SemaphoreType.DMA((N_DEV,)),
        ],
        compiler_params=pltpu.CompilerParams(
            collective_id=0, vmem_limit_bytes=100 * 1024 * 1024
        ),
    )(x, w_mat.astype(jnp.bfloat16))
